# baseline (speedup 1.0000x reference)
import functools

import jax
import jax.numpy as jnp
from jax import lax
from jax.experimental import pallas as pl
from jax.experimental.pallas import tpu as pltpu
from jax.experimental.pallas import tpu_sc as plsc

D = 1024
TOT = 4 * 8192
NC, NS = 2, 16
NW = NC * NS
PER_W = TOT // NW
C = 32
NCHUNK = PER_W // C
NBUF = 3

_mesh = plsc.VectorSubcoreMesh(
    core_axis_name="c", subcore_axis_name="s", num_cores=NC, num_subcores=NS
)


@functools.partial(
    pl.kernel,
    mesh=_mesh,
    out_type=jax.ShapeDtypeStruct((TOT, D), jnp.float32),
    scratch_types=[
        pltpu.VMEM((NCHUNK, C), jnp.int32),
        *([pltpu.VMEM((C, D), jnp.float32)] * NBUF),
        *([pltpu.SemaphoreType.DMA] * NBUF),
        *([pltpu.SemaphoreType.DMA] * NBUF),
    ],
)
def _gather_rows(idx_hbm, table_hbm, out_hbm, idx_v, *bufs_sems):
    bufs = bufs_sems[:NBUF]
    gsems = bufs_sems[NBUF : 2 * NBUF]
    wsems = bufs_sems[2 * NBUF :]

    cid = lax.axis_index("c")
    sid = lax.axis_index("s")
    wid = sid * NC + cid
    base = wid * PER_W

    pltpu.sync_copy(idx_hbm.at[wid], idx_v)

    def gather(i):
        b = i % NBUF
        return pltpu.make_async_copy(
            table_hbm.at[idx_v.at[i]], bufs[b], gsems[b]
        )

    def write(i):
        b = i % NBUF
        return pltpu.make_async_copy(
            bufs[b], out_hbm.at[pl.ds(base + i * C, C)], wsems[b]
        )

    for i in range(NBUF):
        gather(i).start()
    for i in range(NCHUNK):
        gather(i).wait()
        write(i).start()
        j = i + NBUF
        if j < NCHUNK:
            write(i).wait()
            gather(j).start()
    for i in range(NCHUNK - NBUF, NCHUNK):
        write(i).wait()


def kernel(pos_idx, pe):
    idx = pos_idx.astype(jnp.int32).reshape(NW, NCHUNK, C)
    out = _gather_rows(idx, pe)
    return out.reshape(pos_idx.shape + (D,))


# speedup vs baseline: 1.0050x; 1.0050x over previous
import functools

import jax
import jax.numpy as jnp
from jax import lax
from jax.experimental import pallas as pl
from jax.experimental.pallas import tpu as pltpu
from jax.experimental.pallas import tpu_sc as plsc

D = 1024
TOT = 4 * 8192
NC, NS = 2, 16
NW = NC * NS
PER_W = TOT // NW
C = 16
NCHUNK = PER_W // C
NBUF = 7

_mesh = plsc.VectorSubcoreMesh(
    core_axis_name="c", subcore_axis_name="s", num_cores=NC, num_subcores=NS
)


@functools.partial(
    pl.kernel,
    mesh=_mesh,
    out_type=jax.ShapeDtypeStruct((TOT, D), jnp.float32),
    scratch_types=[
        pltpu.VMEM((NCHUNK, C), jnp.int32),
        *([pltpu.VMEM((C, D), jnp.float32)] * NBUF),
        *([pltpu.SemaphoreType.DMA] * NBUF),
        *([pltpu.SemaphoreType.DMA] * NBUF),
    ],
)
def _gather_rows(idx_hbm, table_hbm, out_hbm, idx_v, *bufs_sems):
    bufs = bufs_sems[:NBUF]
    gsems = bufs_sems[NBUF : 2 * NBUF]
    wsems = bufs_sems[2 * NBUF :]

    cid = lax.axis_index("c")
    sid = lax.axis_index("s")
    wid = sid * NC + cid
    base = wid * PER_W

    pltpu.sync_copy(idx_hbm.at[wid], idx_v)

    def gather(i):
        b = i % NBUF
        return pltpu.make_async_copy(
            table_hbm.at[idx_v.at[i]], bufs[b], gsems[b]
        )

    def write(i):
        b = i % NBUF
        return pltpu.make_async_copy(
            bufs[b], out_hbm.at[pl.ds(base + i * C, C)], wsems[b]
        )

    for i in range(NBUF):
        gather(i).start()
    for i in range(NCHUNK):
        gather(i).wait()
        write(i).start()
        j = i + NBUF
        if j < NCHUNK:
            write(i).wait()
            gather(j).start()
    for i in range(NCHUNK - NBUF, NCHUNK):
        write(i).wait()


def kernel(pos_idx, pe):
    idx = pos_idx.astype(jnp.int32).reshape(NW, NCHUNK, C)
    out = _gather_rows(idx, pe)
    return out.reshape(pos_idx.shape + (D,))
